# baseline (speedup 1.0000x reference)
import functools

import jax
import jax.numpy as jnp
from jax.experimental import pallas as pl

_EPS = 1e-12


def _fused_kernel(x_ref, tf_ref, pos_ref, tab_ref, w_ref, b_ref, out_ref):
    x = x_ref[0]
    p = pos_ref[...]
    tf = tf_ref[0, 0, 0]
    t0 = tab_ref[0]
    t1 = tab_ref[1]
    e = x + p + t0[None, :] + tf[:, None] * (t1 - t0)[None, :]
    mean = jnp.mean(e, axis=1, keepdims=True)
    c = e - mean
    var = jnp.mean(c * c, axis=1, keepdims=True)
    y = c * jax.lax.rsqrt(var + _EPS) * w_ref[0][None, :] + b_ref[0][None, :]
    out_ref[0] = y


@functools.partial(jax.jit, static_argnames=("block_s",))
def _run(x, typef, pos, tab, w, b, block_s=512):
    B, S, H = x.shape
    nj = S // block_s
    typef4 = typef.reshape(B, nj, 1, block_s)
    grid = (nj, B)
    return pl.pallas_call(
        _fused_kernel,
        grid=grid,
        in_specs=[
            pl.BlockSpec((1, block_s, H), lambda j, bb: (bb, j, 0)),
            pl.BlockSpec((1, 1, 1, block_s), lambda j, bb: (bb, j, 0, 0)),
            pl.BlockSpec((block_s, H), lambda j, bb: (j, 0)),
            pl.BlockSpec((2, H), lambda j, bb: (0, 0)),
            pl.BlockSpec((1, H), lambda j, bb: (0, 0)),
            pl.BlockSpec((1, H), lambda j, bb: (0, 0)),
        ],
        out_specs=pl.BlockSpec((1, block_s, H), lambda j, bb: (bb, j, 0)),
        out_shape=jax.ShapeDtypeStruct((B, S, H), x.dtype),
    )(x, typef4, pos, tab, w, b)


def kernel(concat_embeddings, concat_type, position_table, token_type_table, ln_weight, ln_bias):
    typef = concat_type.astype(jnp.float32)
    w = ln_weight.reshape(1, -1)
    b = ln_bias.reshape(1, -1)
    return _run(concat_embeddings, typef, position_table, token_type_table, w, b, block_s=256)


# speedup vs baseline: 1.4280x; 1.4280x over previous
import functools

import jax
import jax.numpy as jnp
from jax.experimental import pallas as pl

_EPS = 1e-12


def _fused_kernel(x_ref, tf_ref, pos_ref, tab_ref, w_ref, b_ref, out_ref):
    x = x_ref[0]
    p = pos_ref[...]
    tf = tf_ref[0, 0, 0]
    t0 = tab_ref[0]
    t1 = tab_ref[1]
    e = x + p + t0[None, :] + tf[:, None] * (t1 - t0)[None, :]
    mean = jnp.mean(e, axis=1, keepdims=True)
    c = e - mean
    var = jnp.mean(c * c, axis=1, keepdims=True)
    y = c * jax.lax.rsqrt(var + _EPS) * w_ref[0][None, :] + b_ref[0][None, :]
    out_ref[0] = y


@functools.partial(jax.jit, static_argnames=("block_s",))
def _run(x, typef, pos, tab, w, b, block_s=512):
    B, S, H = x.shape
    nj = S // block_s
    typef4 = typef.reshape(B, nj, 1, block_s)
    grid = (nj, B)
    return pl.pallas_call(
        _fused_kernel,
        grid=grid,
        in_specs=[
            pl.BlockSpec((1, block_s, H), lambda j, bb: (bb, j, 0)),
            pl.BlockSpec((1, 1, 1, block_s), lambda j, bb: (bb, j, 0, 0)),
            pl.BlockSpec((block_s, H), lambda j, bb: (j, 0)),
            pl.BlockSpec((2, H), lambda j, bb: (0, 0)),
            pl.BlockSpec((1, H), lambda j, bb: (0, 0)),
            pl.BlockSpec((1, H), lambda j, bb: (0, 0)),
        ],
        out_specs=pl.BlockSpec((1, block_s, H), lambda j, bb: (bb, j, 0)),
        out_shape=jax.ShapeDtypeStruct((B, S, H), x.dtype),
    )(x, typef4, pos, tab, w, b)


def kernel(concat_embeddings, concat_type, position_table, token_type_table, ln_weight, ln_bias):
    typef = concat_type.astype(jnp.float32)
    w = ln_weight.reshape(1, -1)
    b = ln_bias.reshape(1, -1)
    return _run(concat_embeddings, typef, position_table, token_type_table, w, b, block_s=1024)


# speedup vs baseline: 1.4873x; 1.0415x over previous
import functools

import jax
import jax.numpy as jnp
from jax.experimental import pallas as pl

_EPS = 1e-12


def _fused_kernel(x_ref, tf_ref, pos_ref, tab_ref, w_ref, b_ref, out_ref):
    x = x_ref[0]
    p = pos_ref[...]
    tf = tf_ref[0, 0, 0]
    t0 = tab_ref[0]
    t1 = tab_ref[1]
    e = x + p + t0[None, :] + tf[:, None] * (t1 - t0)[None, :]
    mean = jnp.mean(e, axis=1, keepdims=True)
    c = e - mean
    var = jnp.mean(c * c, axis=1, keepdims=True)
    y = c * jax.lax.rsqrt(var + _EPS) * w_ref[0][None, :] + b_ref[0][None, :]
    out_ref[0] = y


@functools.partial(jax.jit, static_argnames=("block_s",))
def _run(x, typef, pos, tab, w, b, block_s=512):
    B, S, H = x.shape
    nj = S // block_s
    typef4 = typef.reshape(B, nj, 1, block_s)
    grid = (nj, B)
    return pl.pallas_call(
        _fused_kernel,
        grid=grid,
        in_specs=[
            pl.BlockSpec((1, block_s, H), lambda j, bb: (bb, j, 0)),
            pl.BlockSpec((1, 1, 1, block_s), lambda j, bb: (bb, j, 0, 0)),
            pl.BlockSpec((block_s, H), lambda j, bb: (j, 0)),
            pl.BlockSpec((2, H), lambda j, bb: (0, 0)),
            pl.BlockSpec((1, H), lambda j, bb: (0, 0)),
            pl.BlockSpec((1, H), lambda j, bb: (0, 0)),
        ],
        out_specs=pl.BlockSpec((1, block_s, H), lambda j, bb: (bb, j, 0)),
        out_shape=jax.ShapeDtypeStruct((B, S, H), x.dtype),
    )(x, typef4, pos, tab, w, b)


def kernel(concat_embeddings, concat_type, position_table, token_type_table, ln_weight, ln_bias):
    typef = concat_type.astype(jnp.float32)
    w = ln_weight.reshape(1, -1)
    b = ln_bias.reshape(1, -1)
    return _run(concat_embeddings, typef, position_table, token_type_table, w, b, block_s=2048)
